# Initial kernel scaffold; baseline (speedup 1.0000x reference)
import jax
import jax.numpy as jnp
from jax.experimental import pallas as pl


def kernel(x_1, x_2_pos, edge_index_1, edge_index_2_pos, edge_attr_1, edge_attr_2_pos, params):
    raise NotImplementedError("write your pallas kernel here")


# speedup vs baseline: 125.8892x; 125.8892x over previous
import math

import jax
import jax.numpy as jnp
from jax import lax
from jax.experimental import pallas as pl
from jax.experimental.pallas import tpu as pltpu

_H = 4
_CH = 300
_CHP = 384
_N = 200
_E = 3200
_SCALE = 1.0 / math.sqrt(_CH)
_NEG = -1e30


def _adj(ei_ref):
    src = ei_ref[0:1, :]
    dst = ei_ref[1:2, :]
    rows = lax.broadcasted_iota(jnp.int32, (_N, _E), 0)
    d_oh = (rows == dst).astype(jnp.float32)
    s_oh = (rows == src).astype(jnp.float32)
    return lax.dot_general(d_oh, s_oh, (((1,), (1,)), ((), ())),
                           preferred_element_type=jnp.float32)


def _proj(x, w_ref, b_ref):
    return jnp.dot(x, w_ref[...], preferred_element_type=jnp.float32) + b_ref[...]


def _attn_masked(q, k, v, a):
    mask = a > 0.0
    acc = jnp.zeros((_N, _CHP), jnp.float32)
    for h in range(_H):
        sl = slice(h * _CHP, (h + 1) * _CHP)
        qh, kh, vh = q[:, sl], k[:, sl], v[:, sl]
        s = lax.dot_general(qh, kh, (((1,), (1,)), ((), ())),
                            preferred_element_type=jnp.float32) * _SCALE
        sm = jnp.where(mask, s, _NEG)
        amax = jnp.max(sm, axis=1, keepdims=True)
        amax = jnp.where(amax <= _NEG * 0.5, 0.0, amax)
        ex = a * jnp.exp(jnp.where(mask, s - amax, _NEG))
        den = jnp.sum(ex, axis=1, keepdims=True)
        o = jnp.dot(ex, vh, preferred_element_type=jnp.float32) / (den + 1e-16)
        acc = acc + o
    return acc * (1.0 / _H)


def _attn_dense(q, k, v):
    acc = jnp.zeros((_N, _CHP), jnp.float32)
    for h in range(_H):
        sl = slice(h * _CHP, (h + 1) * _CHP)
        qh, kh, vh = q[:, sl], k[:, sl], v[:, sl]
        s = lax.dot_general(qh, kh, (((1,), (1,)), ((), ())),
                            preferred_element_type=jnp.float32) * _SCALE
        amax = jnp.max(s, axis=1, keepdims=True)
        ex = jnp.exp(s - amax)
        den = jnp.sum(ex, axis=1, keepdims=True)
        o = jnp.dot(ex, vh, preferred_element_type=jnp.float32) / (den + 1e-16)
        acc = acc + o
    return acc * (1.0 / _H)


def _conv_self(x, a, wq, bq, wk, bk, wv, bv, ws, bs):
    q = _proj(x, wq, bq)
    k = _proj(x, wk, bk)
    v = _proj(x, wv, bv)
    o = _attn_masked(q, k, v, a)[:, :_CH]
    return o + _proj(x, ws, bs)


def _conv_cross(xd, xs, wq, bq, wk, bk, wv, bv, ws, bs):
    q = _proj(xd, wq, bq)
    k = _proj(xs, wk, bk)
    v = _proj(xs, wv, bv)
    o = _attn_dense(q, k, v)[:, :_CH]
    return o + _proj(xd, ws, bs)


def _l2norm(x):
    n = jnp.sqrt(jnp.sum(x * x, axis=1, keepdims=True))
    return x / jnp.maximum(n, 1e-12)


def _body(x1_ref, x2_ref, ei1_ref, ei2_ref, *refs):
    cp = refs[:32]
    mp = refs[32:40]
    x1p_ref, x2p_ref, out_ref = refs[40:]

    x1 = x1_ref[...]
    x2 = x2_ref[...]
    a1 = _adj(ei1_ref)
    a2 = _adj(ei2_ref)

    x1 = _conv_self(x1, a1, *cp[0:8])
    x2 = _conv_self(x2, a2, *cp[8:16])
    x1c = _conv_cross(x1, x2, *cp[16:24])
    x2c = _conv_cross(x2, x1, *cp[24:32])
    x1n = _l2norm(x1c)
    x2n = _l2norm(x2c)

    x1p = jnp.mean(x1n, axis=0, keepdims=True)
    x2p = jnp.mean(x2n, axis=0, keepdims=True)
    h = jnp.concatenate([x1p, x2p], axis=1)
    h = jnp.maximum(jnp.dot(h, mp[0][...], preferred_element_type=jnp.float32)
                    + mp[1][...], 0.0)
    h = jnp.maximum(jnp.dot(h, mp[2][...], preferred_element_type=jnp.float32)
                    + mp[3][...], 0.0)
    h = jnp.maximum(jnp.dot(h, mp[4][...], preferred_element_type=jnp.float32)
                    + mp[5][...], 0.0)
    z = jnp.dot(h, mp[6][...], preferred_element_type=jnp.float32) + mp[7][...]
    o = 1.0 / (1.0 + jnp.exp(-z))

    x1p_ref[...] = x1p
    x2p_ref[...] = x2p
    out_ref[...] = o


def _pad_head_w(w):
    w = w.reshape(_CH, _H, _CH)
    w = jnp.pad(w, ((0, 0), (0, 0), (0, _CHP - _CH)))
    return w.reshape(_CH, _H * _CHP)


def _pad_head_b(b):
    b = b.reshape(_H, _CH)
    b = jnp.pad(b, ((0, 0), (0, _CHP - _CH)))
    return b.reshape(1, _H * _CHP)


def _conv_args(p):
    return [_pad_head_w(p['Wq']), _pad_head_b(p['bq']),
            _pad_head_w(p['Wk']), _pad_head_b(p['bk']),
            _pad_head_w(p['Wv']), _pad_head_b(p['bv']),
            p['Ws'], p['bs'].reshape(1, _CH)]


def kernel(x_1, x_2_pos, edge_index_1, edge_index_2_pos, edge_attr_1,
           edge_attr_2_pos, params):
    lp = params['layers'][0]
    m = params['mlp']
    args = [x_1, x_2_pos,
            edge_index_1.astype(jnp.int32), edge_index_2_pos.astype(jnp.int32)]
    for name in ('text_self', 'graph_self', 'text_cross', 'graph_cross'):
        args.extend(_conv_args(lp[name]))
    args.extend([m['W1'], m['b1'].reshape(1, -1),
                 m['W2'], m['b2'].reshape(1, -1),
                 m['W3'], m['b3'].reshape(1, -1),
                 m['W4'], m['b4'].reshape(1, -1)])

    x1p, x2p, out = pl.pallas_call(
        _body,
        out_shape=[
            jax.ShapeDtypeStruct((1, _CH), jnp.float32),
            jax.ShapeDtypeStruct((1, _CH), jnp.float32),
            jax.ShapeDtypeStruct((1, 1), jnp.float32),
        ],
        compiler_params=pltpu.CompilerParams(
            vmem_limit_bytes=100 * 1024 * 1024),
    )(*args)
    return x1p.reshape(_CH), x2p.reshape(_CH), out.reshape(1)


# speedup vs baseline: 193.7799x; 1.5393x over previous
import math

import jax
import jax.numpy as jnp
from jax import lax
from jax.experimental import pallas as pl
from jax.experimental.pallas import tpu as pltpu

_H = 4
_CH = 300
_N = 200
_E = 3200
_SCALE = 1.0 / math.sqrt(_CH)
_NEG = -1e30


def _adj(ei_ref):
    src = ei_ref[0:1, :]
    dst = ei_ref[1:2, :]
    rows = lax.broadcasted_iota(jnp.int32, (_N, _E), 0)
    d_oh = (rows == dst).astype(jnp.bfloat16)
    s_oh = (rows == src).astype(jnp.bfloat16)
    return lax.dot_general(d_oh, s_oh, (((1,), (1,)), ((), ())),
                           preferred_element_type=jnp.float32)


def _proj(x, w_ref, b_ref):
    return jnp.dot(x, w_ref[...], preferred_element_type=jnp.float32) + b_ref[...]


def _attn_masked(q, k, v, a):
    mask = a > 0.0
    acc = jnp.zeros((_N, _CH), jnp.float32)
    for h in range(_H):
        sl = slice(h * _CH, (h + 1) * _CH)
        qh, kh, vh = q[:, sl], k[:, sl], v[:, sl]
        s = lax.dot_general(qh, kh, (((1,), (1,)), ((), ())),
                            preferred_element_type=jnp.float32) * _SCALE
        sm = jnp.where(mask, s, _NEG)
        amax = jnp.max(sm, axis=1, keepdims=True)
        amax = jnp.where(amax <= _NEG * 0.5, 0.0, amax)
        ex = a * jnp.exp(jnp.where(mask, s - amax, _NEG))
        den = jnp.sum(ex, axis=1, keepdims=True)
        o = jnp.dot(ex, vh, preferred_element_type=jnp.float32) / (den + 1e-16)
        acc = acc + o
    return acc * (1.0 / _H)


def _attn_dense(q, k, v):
    acc = jnp.zeros((_N, _CH), jnp.float32)
    for h in range(_H):
        sl = slice(h * _CH, (h + 1) * _CH)
        qh, kh, vh = q[:, sl], k[:, sl], v[:, sl]
        s = lax.dot_general(qh, kh, (((1,), (1,)), ((), ())),
                            preferred_element_type=jnp.float32) * _SCALE
        amax = jnp.max(s, axis=1, keepdims=True)
        ex = jnp.exp(s - amax)
        den = jnp.sum(ex, axis=1, keepdims=True)
        o = jnp.dot(ex, vh, preferred_element_type=jnp.float32) / (den + 1e-16)
        acc = acc + o
    return acc * (1.0 / _H)


def _conv_self(x, a, wq, bq, wk, bk, wv, bv, ws, bs):
    q = _proj(x, wq, bq)
    k = _proj(x, wk, bk)
    v = _proj(x, wv, bv)
    o = _attn_masked(q, k, v, a)
    return o + _proj(x, ws, bs)


def _conv_cross(xd, xs, wq, bq, wk, bk, wv, bv, ws, bs):
    q = _proj(xd, wq, bq)
    k = _proj(xs, wk, bk)
    v = _proj(xs, wv, bv)
    o = _attn_dense(q, k, v)
    return o + _proj(xd, ws, bs)


def _l2norm(x):
    n = jnp.sqrt(jnp.sum(x * x, axis=1, keepdims=True))
    return x / jnp.maximum(n, 1e-12)


def _body(x1_ref, x2_ref, ei1_ref, ei2_ref, *refs):
    cp = refs[:32]
    mp = refs[32:40]
    x1p_ref, x2p_ref, out_ref = refs[40:]

    x1 = x1_ref[...]
    x2 = x2_ref[...]
    a1 = _adj(ei1_ref)
    a2 = _adj(ei2_ref)

    x1 = _conv_self(x1, a1, *cp[0:8])
    x2 = _conv_self(x2, a2, *cp[8:16])
    x1c = _conv_cross(x1, x2, *cp[16:24])
    x2c = _conv_cross(x2, x1, *cp[24:32])
    x1n = _l2norm(x1c)
    x2n = _l2norm(x2c)

    x1p = jnp.mean(x1n, axis=0, keepdims=True)
    x2p = jnp.mean(x2n, axis=0, keepdims=True)
    h = jnp.concatenate([x1p, x2p], axis=1)
    h = jnp.maximum(jnp.dot(h, mp[0][...], preferred_element_type=jnp.float32)
                    + mp[1][...], 0.0)
    h = jnp.maximum(jnp.dot(h, mp[2][...], preferred_element_type=jnp.float32)
                    + mp[3][...], 0.0)
    h = jnp.maximum(jnp.dot(h, mp[4][...], preferred_element_type=jnp.float32)
                    + mp[5][...], 0.0)
    z = jnp.dot(h, mp[6][...], preferred_element_type=jnp.float32) + mp[7][...]
    o = 1.0 / (1.0 + jnp.exp(-z))

    x1p_ref[...] = x1p
    x2p_ref[...] = x2p
    out_ref[...] = o


def _conv_args(p):
    return [p['Wq'], p['bq'].reshape(1, -1),
            p['Wk'], p['bk'].reshape(1, -1),
            p['Wv'], p['bv'].reshape(1, -1),
            p['Ws'], p['bs'].reshape(1, _CH)]


def kernel(x_1, x_2_pos, edge_index_1, edge_index_2_pos, edge_attr_1,
           edge_attr_2_pos, params):
    lp = params['layers'][0]
    m = params['mlp']
    args = [x_1, x_2_pos,
            edge_index_1.astype(jnp.int32), edge_index_2_pos.astype(jnp.int32)]
    for name in ('text_self', 'graph_self', 'text_cross', 'graph_cross'):
        args.extend(_conv_args(lp[name]))
    args.extend([m['W1'], m['b1'].reshape(1, -1),
                 m['W2'], m['b2'].reshape(1, -1),
                 m['W3'], m['b3'].reshape(1, -1),
                 m['W4'], m['b4'].reshape(1, -1)])

    x1p, x2p, out = pl.pallas_call(
        _body,
        out_shape=[
            jax.ShapeDtypeStruct((1, _CH), jnp.float32),
            jax.ShapeDtypeStruct((1, _CH), jnp.float32),
            jax.ShapeDtypeStruct((1, 1), jnp.float32),
        ],
        compiler_params=pltpu.CompilerParams(
            vmem_limit_bytes=100 * 1024 * 1024),
    )(*args)
    return x1p.reshape(_CH), x2p.reshape(_CH), out.reshape(1)


# speedup vs baseline: 412.5933x; 2.1292x over previous
import math

import jax
import jax.numpy as jnp
from jax import lax
from jax.experimental import pallas as pl
from jax.experimental.pallas import tpu as pltpu

_H = 4
_CH = 300
_N = 200
_E = 3200
_SCALE = 1.0 / math.sqrt(_CH)
_NEG = -1e30


def _adj(ei_ref):
    src = ei_ref[0:1, :]
    dst = ei_ref[1:2, :]
    rows = lax.broadcasted_iota(jnp.int32, (_N, _E), 0)
    d_oh = (rows == dst).astype(jnp.bfloat16)
    s_oh = (rows == src).astype(jnp.bfloat16)
    return lax.dot_general(d_oh, s_oh, (((1,), (1,)), ((), ())),
                           preferred_element_type=jnp.float32)


def _proj(x, w_ref, b_ref):
    b = b_ref[...].reshape(1, -1)
    return jnp.dot(x, w_ref[...], preferred_element_type=jnp.float32) + b


def _attn_masked(q, k, v, a):
    mask = a > 0.0
    acc = jnp.zeros((_N, _CH), jnp.float32)
    for h in range(_H):
        sl = slice(h * _CH, (h + 1) * _CH)
        qh, kh, vh = q[:, sl], k[:, sl], v[:, sl]
        s = lax.dot_general(qh, kh, (((1,), (1,)), ((), ())),
                            preferred_element_type=jnp.float32) * _SCALE
        sm = jnp.where(mask, s, _NEG)
        amax = jnp.max(sm, axis=1, keepdims=True)
        amax = jnp.where(amax <= _NEG * 0.5, 0.0, amax)
        ex = a * jnp.exp(jnp.where(mask, s - amax, _NEG))
        den = jnp.sum(ex, axis=1, keepdims=True)
        o = jnp.dot(ex, vh, preferred_element_type=jnp.float32) / (den + 1e-16)
        acc = acc + o
    return acc * (1.0 / _H)


def _attn_dense(q, k, v):
    acc = jnp.zeros((_N, _CH), jnp.float32)
    for h in range(_H):
        sl = slice(h * _CH, (h + 1) * _CH)
        qh, kh, vh = q[:, sl], k[:, sl], v[:, sl]
        s = lax.dot_general(qh, kh, (((1,), (1,)), ((), ())),
                            preferred_element_type=jnp.float32) * _SCALE
        amax = jnp.max(s, axis=1, keepdims=True)
        ex = jnp.exp(s - amax)
        den = jnp.sum(ex, axis=1, keepdims=True)
        o = jnp.dot(ex, vh, preferred_element_type=jnp.float32) / (den + 1e-16)
        acc = acc + o
    return acc * (1.0 / _H)


def _conv_self(x, a, wq, bq, wk, bk, wv, bv, ws, bs):
    q = _proj(x, wq, bq)
    k = _proj(x, wk, bk)
    v = _proj(x, wv, bv)
    o = _attn_masked(q, k, v, a)
    return o + _proj(x, ws, bs)


def _conv_cross(xd, xs, wq, bq, wk, bk, wv, bv, ws, bs):
    q = _proj(xd, wq, bq)
    k = _proj(xs, wk, bk)
    v = _proj(xs, wv, bv)
    o = _attn_dense(q, k, v)
    return o + _proj(xd, ws, bs)


def _l2norm(x):
    n = jnp.sqrt(jnp.sum(x * x, axis=1, keepdims=True))
    return x / jnp.maximum(n, 1e-12)


def _body(x1_ref, x2_ref, ei1_ref, ei2_ref, *refs):
    cp = refs[:32]
    mp = refs[32:40]
    x1p_ref, x2p_ref, out_ref = refs[40:]

    x1 = x1_ref[...]
    x2 = x2_ref[...]
    a1 = _adj(ei1_ref)
    a2 = _adj(ei2_ref)

    x1 = _conv_self(x1, a1, *cp[0:8])
    x2 = _conv_self(x2, a2, *cp[8:16])
    x1c = _conv_cross(x1, x2, *cp[16:24])
    x2c = _conv_cross(x2, x1, *cp[24:32])
    x1n = _l2norm(x1c)
    x2n = _l2norm(x2c)

    x1p = jnp.mean(x1n, axis=0, keepdims=True)
    x2p = jnp.mean(x2n, axis=0, keepdims=True)
    h = jnp.concatenate([x1p, x2p], axis=1)
    h = jnp.maximum(jnp.dot(h, mp[0][...], preferred_element_type=jnp.float32)
                    + mp[1][...].reshape(1, -1), 0.0)
    h = jnp.maximum(jnp.dot(h, mp[2][...], preferred_element_type=jnp.float32)
                    + mp[3][...].reshape(1, -1), 0.0)
    h = jnp.maximum(jnp.dot(h, mp[4][...], preferred_element_type=jnp.float32)
                    + mp[5][...].reshape(1, -1), 0.0)
    z = jnp.dot(h, mp[6][...], preferred_element_type=jnp.float32) + mp[7][...].reshape(1, -1)
    o = 1.0 / (1.0 + jnp.exp(-z))

    x1p_ref[...] = x1p.reshape(_CH)
    x2p_ref[...] = x2p.reshape(_CH)
    out_ref[...] = o.reshape(1)


def _conv_args(p):
    return [p['Wq'], p['bq'], p['Wk'], p['bk'],
            p['Wv'], p['bv'], p['Ws'], p['bs']]


def kernel(x_1, x_2_pos, edge_index_1, edge_index_2_pos, edge_attr_1,
           edge_attr_2_pos, params):
    lp = params['layers'][0]
    m = params['mlp']
    args = [x_1, x_2_pos,
            edge_index_1.astype(jnp.int32), edge_index_2_pos.astype(jnp.int32)]
    for name in ('text_self', 'graph_self', 'text_cross', 'graph_cross'):
        args.extend(_conv_args(lp[name]))
    args.extend([m['W1'], m['b1'], m['W2'], m['b2'],
                 m['W3'], m['b3'], m['W4'], m['b4']])

    x1p, x2p, out = pl.pallas_call(
        _body,
        out_shape=[
            jax.ShapeDtypeStruct((_CH,), jnp.float32),
            jax.ShapeDtypeStruct((_CH,), jnp.float32),
            jax.ShapeDtypeStruct((1,), jnp.float32),
        ],
        compiler_params=pltpu.CompilerParams(
            vmem_limit_bytes=100 * 1024 * 1024),
    )(*args)
    return x1p, x2p, out
